# baseline (speedup 1.0000x reference)
import functools

import jax
import jax.numpy as jnp
from jax import lax
from jax.experimental import pallas as pl
from jax.experimental.pallas import tpu as pltpu
from jax.experimental.pallas import tpu_sc as plsc

NE = 64
K = 8
D = 2048
T = 16384
R = 1024
GRID = T // R
ALPHA = 0.001

NC = 2
NS = 16
NW = NC * NS
L = 16
RW = T // NW
NG = RW // L

_AUX_SCALE = ALPHA * NE / (float(T) * float(K) * float(T))

_SORT8 = [(0, 1), (2, 3), (4, 5), (6, 7), (0, 2), (1, 3), (4, 6), (5, 7),
          (1, 2), (5, 6), (0, 4), (1, 5), (2, 6), (3, 7), (2, 4), (3, 5),
          (1, 2), (3, 4), (5, 6)]
_BM8 = [(0, 4), (1, 5), (2, 6), (3, 7), (0, 2), (1, 3), (4, 6), (5, 7),
        (0, 1), (2, 3), (4, 5), (6, 7)]


def _logits_body(x_ref, w_ref, key_ref, ps_ref, pi_ref):
    step = pl.program_id(0)
    logits = lax.dot_general(
        w_ref[...], x_ref[...], (((1,), (1,)), ((), ())),
        preferred_element_type=jnp.float32)
    xi = lax.bitcast_convert_type(logits, jnp.int32)
    neg = lax.shift_right_arithmetic(xi, 31)
    s = (xi ^ (neg & jnp.int32(0x7FFFFFFF))) ^ jnp.int32(-2147483648)
    su = lax.bitcast_convert_type(s, jnp.uint32) >> 1
    row = lax.broadcasted_iota(jnp.uint32, (NE, R), 0)
    key = (su & jnp.uint32(0x7FFFFFC0)) | (jnp.uint32(63) - row)
    key_ref[...] = lax.bitcast_convert_type(key, jnp.float32)
    m = jnp.max(logits, axis=0, keepdims=True)
    e = jnp.exp(logits - m)
    s = jnp.sum(e, axis=0, keepdims=True)
    scores = e / s
    pi_part = jnp.sum(scores.reshape(NE, R // 128, 128), axis=1)

    @pl.when(step == 0)
    def _():
        pi_ref[...] = jnp.zeros_like(pi_ref)

    pi_ref[...] += pi_part

    @pl.when(step == GRID - 1)
    def _():
        ps_ref[...] = pi_ref[...]


def _logits_call(xf, weight):
    return pl.pallas_call(
        _logits_body,
        grid=(GRID,),
        in_specs=[
            pl.BlockSpec((R, D), lambda i: (i, 0)),
            pl.BlockSpec((NE, D), lambda i: (0, 0)),
        ],
        out_specs=[
            pl.BlockSpec((NE, R), lambda i: (0, i)),
            pl.BlockSpec((NE, 128), lambda i: (0, 0)),
        ],
        out_shape=[
            jax.ShapeDtypeStruct((NE, T), jnp.float32),
            jax.ShapeDtypeStruct((NE, 128), jnp.float32),
        ],
        scratch_shapes=[
            pltpu.VMEM((NE, 128), jnp.float32),
        ],
        compiler_params=pltpu.CompilerParams(
            dimension_semantics=("arbitrary",)),
    )(xf, weight)


def _sort_net(keys, net):
    keys = list(keys)
    for i, j in net:
        hi = jnp.maximum(keys[i], keys[j])
        lo = jnp.minimum(keys[i], keys[j])
        keys[i] = hi
        keys[j] = lo
    return keys


@functools.partial(
    pl.kernel,
    out_type=(
        jax.ShapeDtypeStruct((K, T), jnp.float32),
        jax.ShapeDtypeStruct((K, T), jnp.int32),
        jax.ShapeDtypeStruct((NW, NE), jnp.float32),
    ),
    mesh=plsc.VectorSubcoreMesh(core_axis_name="c", subcore_axis_name="s"),
    scratch_types=[
        pltpu.VMEM((NE, RW), jnp.float32),
        pltpu.VMEM((K, RW), jnp.float32),
        pltpu.VMEM((K, RW), jnp.int32),
        pltpu.VMEM((NE,), jnp.float32),
        pltpu.SemaphoreType.DMA,
    ],
    compiler_params=pltpu.CompilerParams(needs_layout_passes=False),
)
def _sc_route(key_hbm, tw_hbm, ti_hbm, cnt_hbm, kv, twv, tiv, cntv, sem):
    wid = lax.axis_index("s") * NC + lax.axis_index("c")
    base = wid * RW
    half = RW // 2
    pltpu.sync_copy(key_hbm.at[:, pl.ds(base, half)],
                    kv.at[:, pl.ds(0, half)])
    cp2 = pltpu.async_copy(key_hbm.at[:, pl.ds(base + half, half)],
                           kv.at[:, pl.ds(half, half)], sem)

    ones16 = jnp.ones((L,), jnp.float32)
    for e in range(0, NE, L):
        cntv[pl.ds(e, L)] = jnp.zeros((L,), jnp.float32)

    def run_group(col):
        load = lambda j: kv[j, pl.ds(col, L)]
        top = _sort_net([load(j) for j in range(K)], _SORT8)
        for b in range(1, NE // K):
            batch = _sort_net([load(b * K + j) for j in range(K)], _SORT8)
            merged = [jnp.maximum(top[k], batch[K - 1 - k]) for k in range(K)]
            top = _sort_net(merged, _BM8)
        tops_u = [lax.bitcast_convert_type(t, jnp.uint32) for t in top]
        idxs = [lax.bitcast_convert_type(jnp.uint32(63) - (u & jnp.uint32(63)),
                                         jnp.int32) for u in tops_u]
        vals = []
        for u in tops_u:
            su = (u & jnp.uint32(0x7FFFFFC0)) << 1
            s2 = lax.bitcast_convert_type(su, jnp.int32) ^ jnp.int32(-2147483648)
            neg = lax.shift_right_arithmetic(s2, 31)
            xi = s2 ^ (neg & jnp.int32(0x7FFFFFFF))
            vals.append(lax.bitcast_convert_type(xi, jnp.float32))
        es = [ones16] + [jnp.exp(v - vals[0]) for v in vals[1:]]
        ssum = es[0]
        for k in range(1, K):
            ssum = ssum + es[k]
        inv = ones16 / ssum
        for k in range(K):
            twv[k, pl.ds(col, L)] = es[k] * inv
            tiv[k, pl.ds(col, L)] = idxs[k]
            plsc.addupdate_scatter(cntv, [idxs[k]], ones16)

    def group_body(g, _):
        run_group(g * L)
        return 0

    lax.fori_loop(0, NG // 2, group_body, 0)
    cp2.wait()
    lax.fori_loop(NG // 2, NG, group_body, 0)
    pltpu.sync_copy(twv, tw_hbm.at[:, pl.ds(base, RW)])
    pltpu.sync_copy(tiv, ti_hbm.at[:, pl.ds(base, RW)])
    pltpu.sync_copy(cntv, cnt_hbm.at[wid])


def kernel(x, weight):
    xf = x.reshape(T, D)
    lg, ps = _logits_call(xf, weight)
    tw, ti, cnt = _sc_route(lg)
    pi_vec = ps.sum(axis=1)
    cnt_vec = cnt.sum(axis=0)
    aux = jnp.sum(pi_vec * cnt_vec) * jnp.float32(_AUX_SCALE)
    return tw.T, ti.T, aux


# speedup vs baseline: 1.0013x; 1.0013x over previous
import functools

import jax
import jax.numpy as jnp
from jax import lax
from jax.experimental import pallas as pl
from jax.experimental.pallas import tpu as pltpu
from jax.experimental.pallas import tpu_sc as plsc

NE = 64
K = 8
D = 2048
T = 16384
R = 1024
GRID = T // R
ALPHA = 0.001

NC = 2
NS = 16
NW = NC * NS
L = 16
RW = T // NW
NG = RW // L

_AUX_SCALE = ALPHA * NE / (float(T) * float(K) * float(T))

_SORT8 = [(0, 1), (2, 3), (4, 5), (6, 7), (0, 2), (1, 3), (4, 6), (5, 7),
          (1, 2), (5, 6), (0, 4), (1, 5), (2, 6), (3, 7), (2, 4), (3, 5),
          (1, 2), (3, 4), (5, 6)]
_BM8 = [(0, 4), (1, 5), (2, 6), (3, 7), (0, 2), (1, 3), (4, 6), (5, 7),
        (0, 1), (2, 3), (4, 5), (6, 7)]


def _logits_body(x_ref, w_ref, key_ref, ps_ref, pi_ref):
    step = pl.program_id(0)
    logits = lax.dot_general(
        w_ref[...], x_ref[...], (((1,), (1,)), ((), ())),
        preferred_element_type=jnp.float32)
    xi = lax.bitcast_convert_type(logits, jnp.int32)
    neg = lax.shift_right_arithmetic(xi, 31)
    s = (xi ^ (neg & jnp.int32(0x7FFFFFFF))) ^ jnp.int32(-2147483648)
    su = lax.bitcast_convert_type(s, jnp.uint32)
    row = lax.broadcasted_iota(jnp.uint32, (NE, R), 0)
    key_ref[...] = (su & jnp.uint32(0xFFFFFFC0)) | (jnp.uint32(63) - row)
    m = jnp.max(logits, axis=0, keepdims=True)
    e = jnp.exp(logits - m)
    s = jnp.sum(e, axis=0, keepdims=True)
    scores = e / s
    pi_part = jnp.sum(scores.reshape(NE, R // 128, 128), axis=1)

    @pl.when(step == 0)
    def _():
        pi_ref[...] = jnp.zeros_like(pi_ref)

    pi_ref[...] += pi_part

    @pl.when(step == GRID - 1)
    def _():
        ps_ref[...] = pi_ref[...]


def _logits_call(xf, weight):
    return pl.pallas_call(
        _logits_body,
        grid=(GRID,),
        in_specs=[
            pl.BlockSpec((R, D), lambda i: (i, 0)),
            pl.BlockSpec((NE, D), lambda i: (0, 0)),
        ],
        out_specs=[
            pl.BlockSpec((NE, R), lambda i: (0, i)),
            pl.BlockSpec((NE, 128), lambda i: (0, 0)),
        ],
        out_shape=[
            jax.ShapeDtypeStruct((NE, T), jnp.uint32),
            jax.ShapeDtypeStruct((NE, 128), jnp.float32),
        ],
        scratch_shapes=[
            pltpu.VMEM((NE, 128), jnp.float32),
        ],
        compiler_params=pltpu.CompilerParams(
            dimension_semantics=("arbitrary",)),
    )(xf, weight)


def _sort_net(keys, net):
    keys = list(keys)
    for i, j in net:
        hi = jnp.maximum(keys[i], keys[j])
        lo = jnp.minimum(keys[i], keys[j])
        keys[i] = hi
        keys[j] = lo
    return keys


@functools.partial(
    pl.kernel,
    out_type=(
        jax.ShapeDtypeStruct((K, T), jnp.float32),
        jax.ShapeDtypeStruct((K, T), jnp.int32),
        jax.ShapeDtypeStruct((NW, NE), jnp.float32),
    ),
    mesh=plsc.VectorSubcoreMesh(core_axis_name="c", subcore_axis_name="s"),
    scratch_types=[
        pltpu.VMEM((NE, RW), jnp.uint32),
        pltpu.VMEM((K, RW), jnp.float32),
        pltpu.VMEM((K, RW), jnp.int32),
        pltpu.VMEM((NE,), jnp.float32),
        pltpu.SemaphoreType.DMA,
    ],
    compiler_params=pltpu.CompilerParams(needs_layout_passes=False),
)
def _sc_route(key_hbm, tw_hbm, ti_hbm, cnt_hbm, kv, twv, tiv, cntv, sem):
    wid = lax.axis_index("s") * NC + lax.axis_index("c")
    base = wid * RW
    half = RW // 2
    pltpu.sync_copy(key_hbm.at[:, pl.ds(base, half)],
                    kv.at[:, pl.ds(0, half)])
    cp2 = pltpu.async_copy(key_hbm.at[:, pl.ds(base + half, half)],
                           kv.at[:, pl.ds(half, half)], sem)

    ones16 = jnp.ones((L,), jnp.float32)
    for e in range(0, NE, L):
        cntv[pl.ds(e, L)] = jnp.zeros((L,), jnp.float32)

    def run_group(col):
        load = lambda j: kv[j, pl.ds(col, L)]
        top = _sort_net([load(j) for j in range(K)], _SORT8)
        for b in range(1, NE // K):
            batch = _sort_net([load(b * K + j) for j in range(K)], _SORT8)
            merged = [jnp.maximum(top[k], batch[K - 1 - k]) for k in range(K)]
            top = _sort_net(merged, _BM8)
        idxs = [lax.bitcast_convert_type(jnp.uint32(63) - (t & jnp.uint32(63)),
                                         jnp.int32) for t in top]
        vals = []
        for t in top:
            si = lax.bitcast_convert_type(t ^ jnp.uint32(0x80000000),
                                          jnp.int32)
            neg = lax.shift_right_arithmetic(si, 31)
            xi = (si & jnp.int32(-64)) ^ (neg & jnp.int32(0x7FFFFFFF))
            vals.append(lax.bitcast_convert_type(xi, jnp.float32))
        es = [ones16] + [jnp.exp(v - vals[0]) for v in vals[1:]]
        ssum = es[0]
        for k in range(1, K):
            ssum = ssum + es[k]
        inv = ones16 / ssum
        for k in range(K):
            twv[k, pl.ds(col, L)] = es[k] * inv
            tiv[k, pl.ds(col, L)] = idxs[k]
            plsc.addupdate_scatter(cntv, [idxs[k]], ones16)

    def group_body(g, _):
        run_group(g * L)
        return 0

    lax.fori_loop(0, NG // 2, group_body, 0)
    cp2.wait()
    lax.fori_loop(NG // 2, NG, group_body, 0)
    pltpu.sync_copy(twv, tw_hbm.at[:, pl.ds(base, RW)])
    pltpu.sync_copy(tiv, ti_hbm.at[:, pl.ds(base, RW)])
    pltpu.sync_copy(cntv, cnt_hbm.at[wid])


def kernel(x, weight):
    xf = x.reshape(T, D)
    lg, ps = _logits_call(xf, weight)
    tw, ti, cnt = _sc_route(lg)
    pi_vec = ps.sum(axis=1)
    cnt_vec = cnt.sum(axis=0)
    aux = jnp.sum(pi_vec * cnt_vec) * jnp.float32(_AUX_SCALE)
    return tw.T, ti.T, aux
